# baseline (speedup 1.0000x reference)
import dataclasses
import functools

import jax
import jax.numpy as jnp
from jax import lax
from jax.experimental import pallas as pl
from jax.experimental.pallas import tpu as pltpu
from jax.experimental.pallas import tpu_sc as plsc

N = 10000
E = 320000
D = 128
HID = 64

NC = 2
NS = 16
LANES = 16

NPAD = 10240
ROWS_PER_TILE = NPAD // NS
E_PER_TILE_ALL = E // (NC * NS)
CH = 64
NCHUNK = 162
NRING = 6
EPAD = NC * NS * NCHUNK * CH

def _sc_params():
    cp = pltpu.CompilerParams()
    if "needs_layout_passes" in pltpu.CompilerParams.__dataclass_fields__:
        cp = dataclasses.replace(cp, needs_layout_passes=False)
    return cp


@functools.lru_cache(maxsize=None)
def _mesh():
    return plsc.VectorSubcoreMesh(core_axis_name="c", subcore_axis_name="s")


_BN = 1000


def _tc1_body(x_ref, wl_ref, bl_ref, wg_ref, mean_ref, xg_ref):
    xb = x_ref[...]
    m = jnp.dot(xb, wl_ref[...], preferred_element_type=jnp.float32) + bl_ref[...]
    nrm = jnp.sqrt(jnp.sum(m * m, axis=1, keepdims=True))
    mean_ref[...] = m / jnp.maximum(nrm, 1e-12) * 1.8
    xg_ref[...] = jnp.dot(xb, wg_ref[...], preferred_element_type=jnp.float32)


def _tc1(x, W_lin, b_lin, W_gwl):
    return pl.pallas_call(
        _tc1_body,
        grid=(N // _BN,),
        in_specs=[
            pl.BlockSpec((_BN, D), lambda i: (i, 0)),
            pl.BlockSpec((D, D), lambda i: (0, 0)),
            pl.BlockSpec((1, D), lambda i: (0, 0)),
            pl.BlockSpec((D, D), lambda i: (0, 0)),
        ],
        out_specs=[
            pl.BlockSpec((_BN, D), lambda i: (i, 0)),
            pl.BlockSpec((_BN, D), lambda i: (i, 0)),
        ],
        out_shape=[
            jax.ShapeDtypeStruct((N, D), jnp.float32),
            jax.ShapeDtypeStruct((N, D), jnp.float32),
        ],
    )(x, W_lin, b_lin, W_gwl)


def _tc2_body(p_ref, dinv_ref):
    deg = jnp.sum(p_ref[...], axis=0, keepdims=True) + 1.0
    dinv_ref[...] = lax.rsqrt(deg)


def _tc2(deg_partials):
    return pl.pallas_call(
        _tc2_body,
        grid=(1,),
        in_specs=[pl.BlockSpec((NC * NS, N), lambda i: (0, 0))],
        out_specs=pl.BlockSpec((1, N), lambda i: (0, 0)),
        out_shape=jax.ShapeDtypeStruct((1, N), jnp.float32),
    )(deg_partials)


def _tc3_body(a_ref, b_ref, c_ref, d_ref, diag_ref, t0_ref, t1_ref):
    dg = diag_ref[...]
    t0_ref[...] = dg * (a_ref[...] + b_ref[...])
    t1_ref[...] = dg * (c_ref[...] + d_ref[...])


def _tc3(p00, p01, p10, p11, diag2d):
    blk = pl.BlockSpec((_BN, D), lambda i: (i, 0))
    return pl.pallas_call(
        _tc3_body,
        grid=(N // _BN,),
        in_specs=[blk, blk, blk, blk, pl.BlockSpec((_BN, 1), lambda i: (i, 0))],
        out_specs=[blk, blk],
        out_shape=[
            jax.ShapeDtypeStruct((N, D), jnp.float32),
            jax.ShapeDtypeStruct((N, D), jnp.float32),
        ],
    )(p00, p01, p10, p11, diag2d)


def _tc4_body(mp0_ref, mp1_ref, mpre_ref, dinv_ref, h00_ref, h01_ref,
              h10_ref, h11_ref, w1_ref, b1_ref, w2_ref, mean_ref, logstd_ref):
    dv = dinv_ref[...]
    mean_ref[...] = mp0_ref[...] + mp1_ref[...] + dv * dv * mpre_ref[...]
    h0 = h00_ref[...] + h01_ref[...]
    h1 = h10_ref[...] + h11_ref[...]
    w1 = w1_ref[...]
    b1 = b1_ref[...]
    w2 = w2_ref[...]
    s0 = jnp.dot(jnp.maximum(jnp.dot(h0, w1, preferred_element_type=jnp.float32)
                             + b1, 0.0), w2, preferred_element_type=jnp.float32)
    s1 = jnp.dot(jnp.maximum(jnp.dot(h1, w1, preferred_element_type=jnp.float32)
                             + b1, 0.0), w2, preferred_element_type=jnp.float32)
    m = jnp.maximum(s0, s1)
    e0 = jnp.exp(s0 - m)
    e1 = jnp.exp(s1 - m)
    inv = 1.0 / (e0 + e1)
    logstd_ref[...] = (e0 * inv) * h0 + (e1 * inv) * h1


def _tc4(mp0, mp1, mean_pre, dinv2d, h00, h01, h10, h11, att_W1, att_b1, att_w2):
    blk = pl.BlockSpec((_BN, D), lambda i: (i, 0))
    col = pl.BlockSpec((_BN, 1), lambda i: (i, 0))
    return pl.pallas_call(
        _tc4_body,
        grid=(N // _BN,),
        in_specs=[blk, blk, blk, col, blk, blk, blk, blk,
                  pl.BlockSpec((D, HID), lambda i: (0, 0)),
                  pl.BlockSpec((1, HID), lambda i: (0, 0)),
                  pl.BlockSpec((HID, 1), lambda i: (0, 0))],
        out_specs=[blk, blk],
        out_shape=[
            jax.ShapeDtypeStruct((N, D), jnp.float32),
            jax.ShapeDtypeStruct((N, D), jnp.float32),
        ],
    )(mp0, mp1, mean_pre, dinv2d, h00, h01, h10, h11, att_W1, att_b1, att_w2)


@functools.lru_cache(maxsize=None)
def _deg_kernel_fn():
    return functools.partial(
        pl.kernel,
        out_type=jax.ShapeDtypeStruct((NC * NS * N,), jnp.float32),
        mesh=_mesh(),
        compiler_params=_sc_params(),
        scratch_types=[
            pltpu.VMEM((N,), jnp.float32),
            pltpu.VMEM((E_PER_TILE_ALL,), jnp.int32),
        ],
    )(_deg_body)


def _deg_body(dst_hbm, out_hbm, dbuf, ibuf):
    cid = lax.axis_index("c")
    sid = lax.axis_index("s")
    wid = cid * NS + sid

    @pl.loop(0, N // LANES)
    def _zero(i):
        dbuf[pl.ds(i * LANES, LANES)] = jnp.zeros((LANES,), jnp.float32)

    pltpu.sync_copy(dst_hbm.at[pl.ds(wid * E_PER_TILE_ALL, E_PER_TILE_ALL)], ibuf)
    ones = jnp.ones((LANES,), jnp.float32)

    @pl.loop(0, E_PER_TILE_ALL // LANES)
    def _hist(i):
        idx = ibuf[pl.ds(i * LANES, LANES)]
        plsc.addupdate_scatter(dbuf, [idx], ones)

    pltpu.sync_copy(dbuf, out_hbm.at[pl.ds(wid * N, N)])


@functools.lru_cache(maxsize=None)
def _w_kernel_fn():
    return functools.partial(
        pl.kernel,
        out_type=jax.ShapeDtypeStruct((E,), jnp.float32),
        mesh=_mesh(),
        compiler_params=_sc_params(),
        scratch_types=[
            pltpu.VMEM((N,), jnp.float32),
            pltpu.VMEM((E_PER_TILE_ALL,), jnp.int32),
            pltpu.VMEM((E_PER_TILE_ALL,), jnp.int32),
            pltpu.VMEM((E_PER_TILE_ALL,), jnp.float32),
        ],
    )(_w_body)


def _w_body(src_hbm, dst_hbm, dinv_hbm, out_hbm, dv, sbuf, dbuf, wbuf):
    cid = lax.axis_index("c")
    sid = lax.axis_index("s")
    wid = cid * NS + sid
    base = wid * E_PER_TILE_ALL
    pltpu.sync_copy(dinv_hbm, dv)
    pltpu.sync_copy(src_hbm.at[pl.ds(base, E_PER_TILE_ALL)], sbuf)
    pltpu.sync_copy(dst_hbm.at[pl.ds(base, E_PER_TILE_ALL)], dbuf)

    @pl.loop(0, E_PER_TILE_ALL // LANES)
    def _w(i):
        sl = pl.ds(i * LANES, LANES)
        a = plsc.load_gather(dv, [sbuf[sl]])
        b = plsc.load_gather(dv, [dbuf[sl]])
        wbuf[sl] = a * b

    pltpu.sync_copy(wbuf, out_hbm.at[pl.ds(base, E_PER_TILE_ALL)])


@functools.lru_cache(maxsize=None)
def _spmm_kernel_fn():
    return functools.partial(
        pl.kernel,
        out_type=jax.ShapeDtypeStruct((NC, NPAD, D), jnp.float32),
        mesh=_mesh(),
        compiler_params=_sc_params(),
        scratch_types=[
            pltpu.VMEM_SHARED((NPAD, D), jnp.float32),
            pltpu.VMEM((NRING, 1, CH), jnp.int32),
            pltpu.VMEM((NRING, 1, CH), jnp.int32),
            pltpu.VMEM((NRING, 1, CH), jnp.float32),
            pltpu.VMEM((2, CH, D), jnp.float32),
            pltpu.VMEM((2, CH, D), jnp.float32),
            pltpu.VMEM((32, D), jnp.float32),
        ] + [pltpu.SemaphoreType.DMA] * (NRING + 4),
    )(_spmm_body)


def _spmm_body(rows_hbm, cols_hbm, vals_hbm, dense_hbm, out_hbm,
               acc, ridx, cidx, vbuf, gbuf, obuf, zbuf, *sems):
    isems = sems[:NRING]
    gsems = sems[NRING:NRING + 2]
    ssems = sems[NRING + 2:NRING + 4]
    cid = lax.axis_index("c")
    sid = lax.axis_index("s")
    wid = cid * NS + sid

    @pl.loop(0, 32)
    def _zfill(r):
        for k in range(D // LANES):
            zbuf[r, pl.ds(k * LANES, LANES)] = jnp.zeros((LANES,), jnp.float32)

    for j in range(ROWS_PER_TILE // 32):
        pltpu.sync_copy(zbuf, acc.at[pl.ds(sid * ROWS_PER_TILE + j * 32, 32)])

    def _issue_idx(ic, k):
        base = wid * NCHUNK + ic
        pltpu.async_copy(rows_hbm.at[base], ridx.at[k], isems[k])
        pltpu.async_copy(cols_hbm.at[base], cidx.at[k], isems[k])
        pltpu.async_copy(vals_hbm.at[base], vbuf.at[k], isems[k])

    def _wait_idx(ic, k):
        base = wid * NCHUNK + ic
        pltpu.make_async_copy(rows_hbm.at[base], ridx.at[k], isems[k]).wait()
        pltpu.make_async_copy(cols_hbm.at[base], cidx.at[k], isems[k]).wait()
        pltpu.make_async_copy(vals_hbm.at[base], vbuf.at[k], isems[k]).wait()

    def _issue_gather(k, b):
        pltpu.async_copy(dense_hbm.at[cidx.at[k, 0]], gbuf.at[b], gsems[b])

    def _wait_gather(k, b):
        pltpu.make_async_copy(dense_hbm.at[cidx.at[k, 0]], gbuf.at[b],
                              gsems[b]).wait()

    def _issue_scat(k, b):
        pltpu.async_copy(obuf.at[b], acc.at[ridx.at[k, 0]], ssems[b], add=True)

    def _wait_scat(k, b):
        pltpu.make_async_copy(obuf.at[b], acc.at[ridx.at[k, 0]], ssems[b]).wait()

    def _mul(k, b):
        @pl.loop(0, CH)
        def _scale(r):
            bidx = jnp.full((LANES,), r, jnp.int32)
            bv = plsc.load_gather(vbuf.at[k, 0], [bidx])
            for f in range(D // LANES):
                obuf[b, r, pl.ds(f * LANES, LANES)] = (
                    gbuf[b, r, pl.ds(f * LANES, LANES)] * bv)

    def _step(i, ic, *, sswait, idx_ahead, gather_ahead):
        k = i % NRING
        b = i % 2
        _wait_gather(k, b)
        if sswait:
            pass
        if idx_ahead:
            _issue_idx(ic + 4, (i + 4) % NRING)
        if gather_ahead:
            _wait_idx(ic + 2, (i + 2) % NRING)
            _issue_gather((i + 2) % NRING, b)

    plsc.subcore_barrier()

    for i in range(4):
        _issue_idx(i, i)
    for i in range(2):
        _wait_idx(i, i)
        _issue_gather(i, i)

    _step(0, 0, sswait=False, idx_ahead=True, gather_ahead=True)
    _step(1, 1, sswait=False, idx_ahead=True, gather_ahead=True)

    @pl.loop(0, (NCHUNK - 6) // 6)
    def _macro(m):
        ic0 = 6 * m + 2
        for kk in range(6):
            _step(kk + 2, ic0 + kk, sswait=True, idx_ahead=True,
                  gather_ahead=True)

    _step(NCHUNK - 4, NCHUNK - 4, sswait=True, idx_ahead=False,
          gather_ahead=True)
    _step(NCHUNK - 3, NCHUNK - 3, sswait=True, idx_ahead=False,
          gather_ahead=True)
    _step(NCHUNK - 2, NCHUNK - 2, sswait=True, idx_ahead=False,
          gather_ahead=False)
    _step(NCHUNK - 1, NCHUNK - 1, sswait=True, idx_ahead=False,
          gather_ahead=False)


    plsc.subcore_barrier()
    pltpu.sync_copy(acc.at[pl.ds(sid * ROWS_PER_TILE, ROWS_PER_TILE)],
                    out_hbm.at[cid, pl.ds(sid * ROWS_PER_TILE, ROWS_PER_TILE)])


def kernel(x, edge_index, phi0_indices, phi0_values, phi1_indices, phi1_values,
           phi_inv0_indices, phi_inv0_values, phi_inv1_indices, phi_inv1_values,
           W_lin, b_lin, W_gwl, diag_w, att_W1, att_b1, att_w2):
    src = edge_index[0]
    dst = edge_index[1]

    mean_pre, xg = _tc1(x, W_lin, b_lin.reshape(1, D), W_gwl)

    deg_partials = _deg_kernel_fn()(dst).reshape(NC * NS, N)
    dinv = _tc2(deg_partials)
    dinv1d = dinv.reshape(N)
    dinv2d = dinv.reshape(N, 1)

    w = _w_kernel_fn()(src, dst, dinv1d)

    def _coo(rows, cols, vals):
        pad = EPAD - rows.shape[0]
        shp = (NC * NS * NCHUNK, 1, CH)
        return (jnp.concatenate([rows, jnp.zeros((pad,), rows.dtype)]).reshape(shp),
                jnp.concatenate([cols, jnp.zeros((pad,), cols.dtype)]).reshape(shp),
                jnp.concatenate([vals, jnp.zeros((pad,), vals.dtype)]).reshape(shp))

    def _spmm(rows, cols, vals, dense):
        return _spmm_kernel_fn()(*_coo(rows, cols, vals), dense)[:, :N]

    p_inv0 = _spmm(phi_inv0_indices[0], phi_inv0_indices[1], phi_inv0_values, xg)
    p_inv1 = _spmm(phi_inv1_indices[0], phi_inv1_indices[1], phi_inv1_values, xg)
    t0, t1 = _tc3(p_inv0[0], p_inv0[1], p_inv1[0], p_inv1[1],
                  diag_w.reshape(N, 1))

    p0 = _spmm(phi0_indices[0], phi0_indices[1], phi0_values, t0)
    p1 = _spmm(phi1_indices[0], phi1_indices[1], phi1_values, t1)

    pm = _spmm(dst, src, w, mean_pre)

    mean, logstd = _tc4(pm[0], pm[1], mean_pre, dinv2d,
                        p0[0], p0[1], p1[0], p1[1],
                        att_W1, att_b1.reshape(1, HID), att_w2)
    return (mean, logstd)


# speedup vs baseline: 3.8055x; 3.8055x over previous
import dataclasses
import functools

import jax
import jax.numpy as jnp
from jax import lax
from jax.experimental import pallas as pl
from jax.experimental.pallas import tpu as pltpu
from jax.experimental.pallas import tpu_sc as plsc

N = 10000
E = 320000
D = 128
HID = 64

NC = 2
NS = 16
LANES = 16

NPAD = 10240
ROWS_PER_TILE = NPAD // NS
E_PER_TILE_ALL = E // (NC * NS)
CH = 80
NCHUNK = 125
EPAD = NC * NS * NCHUNK * CH

def _sc_params():
    cp = pltpu.CompilerParams()
    if "needs_layout_passes" in pltpu.CompilerParams.__dataclass_fields__:
        cp = dataclasses.replace(cp, needs_layout_passes=False)
    return cp


@functools.lru_cache(maxsize=None)
def _mesh():
    return plsc.VectorSubcoreMesh(core_axis_name="c", subcore_axis_name="s")


_BN = 1000


def _tc1_body(x_ref, wl_ref, bl_ref, wg_ref, mean_ref, xg_ref):
    xb = x_ref[...]
    m = jnp.dot(xb, wl_ref[...], preferred_element_type=jnp.float32) + bl_ref[...]
    nrm = jnp.sqrt(jnp.sum(m * m, axis=1, keepdims=True))
    mean_ref[...] = m / jnp.maximum(nrm, 1e-12) * 1.8
    xg_ref[...] = jnp.dot(xb, wg_ref[...], preferred_element_type=jnp.float32)


def _tc1(x, W_lin, b_lin, W_gwl):
    return pl.pallas_call(
        _tc1_body,
        grid=(N // _BN,),
        in_specs=[
            pl.BlockSpec((_BN, D), lambda i: (i, 0)),
            pl.BlockSpec((D, D), lambda i: (0, 0)),
            pl.BlockSpec((1, D), lambda i: (0, 0)),
            pl.BlockSpec((D, D), lambda i: (0, 0)),
        ],
        out_specs=[
            pl.BlockSpec((_BN, D), lambda i: (i, 0)),
            pl.BlockSpec((_BN, D), lambda i: (i, 0)),
        ],
        out_shape=[
            jax.ShapeDtypeStruct((N, D), jnp.float32),
            jax.ShapeDtypeStruct((N, D), jnp.float32),
        ],
    )(x, W_lin, b_lin, W_gwl)


def _tc2_body(p_ref, dinv_ref):
    deg = jnp.sum(p_ref[...], axis=0, keepdims=True) + 1.0
    dinv_ref[...] = lax.rsqrt(deg)


def _tc2(deg_partials):
    return pl.pallas_call(
        _tc2_body,
        grid=(1,),
        in_specs=[pl.BlockSpec((NC * NS, N), lambda i: (0, 0))],
        out_specs=pl.BlockSpec((1, N), lambda i: (0, 0)),
        out_shape=jax.ShapeDtypeStruct((1, N), jnp.float32),
    )(deg_partials)


def _tc3_body(a_ref, b_ref, c_ref, d_ref, diag_ref, t0_ref, t1_ref):
    dg = diag_ref[...]
    t0_ref[...] = dg * (a_ref[...] + b_ref[...])
    t1_ref[...] = dg * (c_ref[...] + d_ref[...])


def _tc3(p00, p01, p10, p11, diag2d):
    blk = pl.BlockSpec((_BN, D), lambda i: (i, 0))
    return pl.pallas_call(
        _tc3_body,
        grid=(N // _BN,),
        in_specs=[blk, blk, blk, blk, pl.BlockSpec((_BN, 1), lambda i: (i, 0))],
        out_specs=[blk, blk],
        out_shape=[
            jax.ShapeDtypeStruct((N, D), jnp.float32),
            jax.ShapeDtypeStruct((N, D), jnp.float32),
        ],
    )(p00, p01, p10, p11, diag2d)


def _tc4_body(mp0_ref, mp1_ref, mpre_ref, dinv_ref, h00_ref, h01_ref,
              h10_ref, h11_ref, w1_ref, b1_ref, w2_ref, mean_ref, logstd_ref):
    dv = dinv_ref[...]
    mean_ref[...] = mp0_ref[...] + mp1_ref[...] + dv * dv * mpre_ref[...]
    h0 = h00_ref[...] + h01_ref[...]
    h1 = h10_ref[...] + h11_ref[...]
    w1 = w1_ref[...]
    b1 = b1_ref[...]
    w2 = w2_ref[...]
    s0 = jnp.dot(jnp.maximum(jnp.dot(h0, w1, preferred_element_type=jnp.float32)
                             + b1, 0.0), w2, preferred_element_type=jnp.float32)
    s1 = jnp.dot(jnp.maximum(jnp.dot(h1, w1, preferred_element_type=jnp.float32)
                             + b1, 0.0), w2, preferred_element_type=jnp.float32)
    m = jnp.maximum(s0, s1)
    e0 = jnp.exp(s0 - m)
    e1 = jnp.exp(s1 - m)
    inv = 1.0 / (e0 + e1)
    logstd_ref[...] = (e0 * inv) * h0 + (e1 * inv) * h1


def _tc4(mp0, mp1, mean_pre, dinv2d, h00, h01, h10, h11, att_W1, att_b1, att_w2):
    blk = pl.BlockSpec((_BN, D), lambda i: (i, 0))
    col = pl.BlockSpec((_BN, 1), lambda i: (i, 0))
    return pl.pallas_call(
        _tc4_body,
        grid=(N // _BN,),
        in_specs=[blk, blk, blk, col, blk, blk, blk, blk,
                  pl.BlockSpec((D, HID), lambda i: (0, 0)),
                  pl.BlockSpec((1, HID), lambda i: (0, 0)),
                  pl.BlockSpec((HID, 1), lambda i: (0, 0))],
        out_specs=[blk, blk],
        out_shape=[
            jax.ShapeDtypeStruct((N, D), jnp.float32),
            jax.ShapeDtypeStruct((N, D), jnp.float32),
        ],
    )(mp0, mp1, mean_pre, dinv2d, h00, h01, h10, h11, att_W1, att_b1, att_w2)


@functools.lru_cache(maxsize=None)
def _deg_kernel_fn():
    return functools.partial(
        pl.kernel,
        out_type=jax.ShapeDtypeStruct((NC * NS * N,), jnp.float32),
        mesh=_mesh(),
        compiler_params=_sc_params(),
        scratch_types=[
            pltpu.VMEM((N,), jnp.float32),
            pltpu.VMEM((E_PER_TILE_ALL,), jnp.int32),
        ],
    )(_deg_body)


def _deg_body(dst_hbm, out_hbm, dbuf, ibuf):
    cid = lax.axis_index("c")
    sid = lax.axis_index("s")
    wid = cid * NS + sid

    @pl.loop(0, N // LANES)
    def _zero(i):
        dbuf[pl.ds(i * LANES, LANES)] = jnp.zeros((LANES,), jnp.float32)

    pltpu.sync_copy(dst_hbm.at[pl.ds(wid * E_PER_TILE_ALL, E_PER_TILE_ALL)], ibuf)
    ones = jnp.ones((LANES,), jnp.float32)

    @pl.loop(0, E_PER_TILE_ALL // LANES)
    def _hist(i):
        idx = ibuf[pl.ds(i * LANES, LANES)]
        plsc.addupdate_scatter(dbuf, [idx], ones)

    pltpu.sync_copy(dbuf, out_hbm.at[pl.ds(wid * N, N)])


@functools.lru_cache(maxsize=None)
def _w_kernel_fn():
    return functools.partial(
        pl.kernel,
        out_type=jax.ShapeDtypeStruct((E,), jnp.float32),
        mesh=_mesh(),
        compiler_params=_sc_params(),
        scratch_types=[
            pltpu.VMEM((N,), jnp.float32),
            pltpu.VMEM((E_PER_TILE_ALL,), jnp.int32),
            pltpu.VMEM((E_PER_TILE_ALL,), jnp.int32),
            pltpu.VMEM((E_PER_TILE_ALL,), jnp.float32),
        ],
    )(_w_body)


def _w_body(src_hbm, dst_hbm, dinv_hbm, out_hbm, dv, sbuf, dbuf, wbuf):
    cid = lax.axis_index("c")
    sid = lax.axis_index("s")
    wid = cid * NS + sid
    base = wid * E_PER_TILE_ALL
    pltpu.sync_copy(dinv_hbm, dv)
    pltpu.sync_copy(src_hbm.at[pl.ds(base, E_PER_TILE_ALL)], sbuf)
    pltpu.sync_copy(dst_hbm.at[pl.ds(base, E_PER_TILE_ALL)], dbuf)

    @pl.loop(0, E_PER_TILE_ALL // LANES)
    def _w(i):
        sl = pl.ds(i * LANES, LANES)
        a = plsc.load_gather(dv, [sbuf[sl]])
        b = plsc.load_gather(dv, [dbuf[sl]])
        wbuf[sl] = a * b

    pltpu.sync_copy(wbuf, out_hbm.at[pl.ds(base, E_PER_TILE_ALL)])


@functools.lru_cache(maxsize=None)
def _spmm_kernel_fn():
    return functools.partial(
        pl.kernel,
        out_type=jax.ShapeDtypeStruct((NC, NPAD, D), jnp.float32),
        mesh=_mesh(),
        compiler_params=_sc_params(),
        scratch_types=[
            pltpu.VMEM_SHARED((NPAD, D), jnp.float32),
            pltpu.VMEM((1, NCHUNK * CH), jnp.int32),
            pltpu.VMEM((1, NCHUNK * CH), jnp.float32),
            pltpu.VMEM((2, 1, CH), jnp.int32),
            pltpu.VMEM((2, 1, CH), jnp.int32),
            pltpu.VMEM((2, CH, D), jnp.float32),
            pltpu.SemaphoreType.DMA,
            pltpu.SemaphoreType.DMA,
        ],
    )(_spmm_body)


def _spmm_body(rc_hbm, vals_hbm, dense_hbm, out_hbm,
               acc, rcbuf, vbuf, rsm, csm, gbuf, gsem0, gsem1):
    gsems = (gsem0, gsem1)
    cid = lax.axis_index("c")
    sid = lax.axis_index("s")
    wid = cid * NS + sid

    @pl.loop(0, CH)
    def _zfill(r):
        for f in range(D // LANES):
            gbuf[0, r, pl.ds(f * LANES, LANES)] = jnp.zeros((LANES,), jnp.float32)

    for j in range(ROWS_PER_TILE // CH):
        pltpu.sync_copy(gbuf.at[0],
                        acc.at[pl.ds(sid * ROWS_PER_TILE + j * CH, CH)])

    pltpu.sync_copy(rc_hbm.at[wid], rcbuf)
    pltpu.sync_copy(vals_hbm.at[wid], vbuf)
    plsc.subcore_barrier()

    def _unpack(i, b):
        @pl.loop(0, CH // LANES)
        def _u(j):
            rc = rcbuf[0, pl.ds(i * CH + j * LANES, LANES)]
            rsm[b, 0, pl.ds(j * LANES, LANES)] = lax.shift_right_logical(rc, 14)
            csm[b, 0, pl.ds(j * LANES, LANES)] = lax.bitwise_and(rc, 16383)

    def _issue_gather(b):
        pltpu.async_copy(dense_hbm.at[csm.at[b, 0]], gbuf.at[b], gsems[b])

    def _wait_gather(b):
        pltpu.make_async_copy(dense_hbm.at[csm.at[b, 0]], gbuf.at[b],
                              gsems[b]).wait()

    def _mul(i, b):
        @pl.loop(0, CH)
        def _scale(r):
            bidx = jnp.full((LANES,), i * CH + r, jnp.int32)
            bv = plsc.load_gather(vbuf.at[0], [bidx])
            for f in range(D // LANES):
                gbuf[b, r, pl.ds(f * LANES, LANES)] = (
                    gbuf[b, r, pl.ds(f * LANES, LANES)] * bv)

    def _step(i, b, prefetch):
        if prefetch:
            _unpack(i + 1, 1 - b)
            _issue_gather(1 - b)
        _wait_gather(b)
        _mul(i, b)
        pltpu.sync_copy(gbuf.at[b], acc.at[rsm.at[b, 0]], add=True)

    _unpack(0, 0)
    _issue_gather(0)

    @pl.loop(0, NCHUNK // 2)
    def _pair(p):
        i0 = 2 * p
        _step(i0, 0, True)
        _step(i0 + 1, 1, True)

    _step(NCHUNK - 1, 0, False)

    plsc.subcore_barrier()
    pltpu.sync_copy(acc.at[pl.ds(sid * ROWS_PER_TILE, ROWS_PER_TILE)],
                    out_hbm.at[cid, pl.ds(sid * ROWS_PER_TILE, ROWS_PER_TILE)])


def kernel(x, edge_index, phi0_indices, phi0_values, phi1_indices, phi1_values,
           phi_inv0_indices, phi_inv0_values, phi_inv1_indices, phi_inv1_values,
           W_lin, b_lin, W_gwl, diag_w, att_W1, att_b1, att_w2):
    src = edge_index[0]
    dst = edge_index[1]

    mean_pre, xg = _tc1(x, W_lin, b_lin.reshape(1, D), W_gwl)

    deg_partials = _deg_kernel_fn()(dst).reshape(NC * NS, N)
    dinv = _tc2(deg_partials)
    dinv1d = dinv.reshape(N)
    dinv2d = dinv.reshape(N, 1)

    w = _w_kernel_fn()(src, dst, dinv1d)

    def _coo(rows, cols, vals):
        shp = (NC * NS, 1, NCHUNK * CH)
        rc = (jnp.left_shift(rows.astype(jnp.int32), 14)
              | cols.astype(jnp.int32)).reshape(shp)
        return (rc, vals.reshape(shp))

    def _spmm(rows, cols, vals, dense):
        return _spmm_kernel_fn()(*_coo(rows, cols, vals), dense)[:, :N]

    p_inv0 = _spmm(phi_inv0_indices[0], phi_inv0_indices[1], phi_inv0_values, xg)
    p_inv1 = _spmm(phi_inv1_indices[0], phi_inv1_indices[1], phi_inv1_values, xg)
    t0, t1 = _tc3(p_inv0[0], p_inv0[1], p_inv1[0], p_inv1[1],
                  diag_w.reshape(N, 1))

    p0 = _spmm(phi0_indices[0], phi0_indices[1], phi0_values, t0)
    p1 = _spmm(phi1_indices[0], phi1_indices[1], phi1_values, t1)

    pm = _spmm(dst, src, w, mean_pre)

    mean, logstd = _tc4(pm[0], pm[1], mean_pre, dinv2d,
                        p0[0], p0[1], p1[0], p1[1],
                        att_W1, att_b1.reshape(1, HID), att_w2)
    return (mean, logstd)
